# baseline (speedup 1.0000x reference)
import jax
import jax.numpy as jnp
from jax import lax
from jax.experimental import pallas as pl
from jax.experimental.pallas import tpu as pltpu

_NBUF = 4
_BI = 200


def _mm(a, b):
    return jax.lax.dot_general(
        a, b, (((1,), (0,)), ((), ())),
        precision=jax.lax.Precision.DEFAULT,
        preferred_element_type=jnp.float32)


def _body(x_ref, w1_ref, adj_ref, b1_ref, w2_ref, b2_ref, o_ref,
          b0, b1v, b2v, b3, ob0, ob1, xv_ref, s1_ref, s2_ref,
          sem0, sem1, sem2, sem3, osem0, osem1, xsem):
    n = adj_ref.shape[0]
    nstrips = n // _BI
    total = 2 * nstrips
    bufs = [b0, b1v, b2v, b3]
    sems = [sem0, sem1, sem2, sem3]
    obufs = [ob0, ob1]
    osems = [osem0, osem1]

    def _strip_copy(s, b):
        r = lax.rem(s, nstrips) * _BI
        return pltpu.make_async_copy(
            adj_ref.at[pl.ds(r, _BI), :], bufs[b], sems[b])

    def _out_copy(s, ob):
        r = lax.rem(s, nstrips) * _BI
        return pltpu.make_async_copy(
            obufs[ob], o_ref.at[pl.ds(r, _BI), :], osems[ob])

    xcopy = pltpu.make_async_copy(x_ref, xv_ref, xsem)
    xcopy.start()
    for b in range(_NBUF):
        _strip_copy(jnp.int32(b), b).start()

    xcopy.wait()
    s1_ref[...] = _mm(xv_ref[...], w1_ref[...])

    def _outer(g, carry):
        for b in range(_NBUF):
            s = _NBUF * g + b
            ob = b % 2
            _strip_copy(s, b).wait()

            @pl.when(s < nstrips)
            def _():
                t = _mm(bufs[b][...], s1_ref[...])
                h = jnp.maximum(t + b1_ref[...], 0.0)
                s2_ref[pl.ds(s * _BI, _BI), :] = _mm(h, w2_ref[...])

            @pl.when(s >= nstrips)
            def _():
                @pl.when(s >= nstrips + 2)
                def _():
                    _out_copy(s - 2, ob).wait()

                t = _mm(bufs[b][...], s2_ref[...])
                logits = t + b2_ref[...]
                m = jnp.max(logits, axis=1, keepdims=True)
                e = jnp.exp(logits - m)
                lse = m + jnp.log(jnp.sum(e, axis=1, keepdims=True))
                obufs[ob][...] = logits - lse
                _out_copy(s, ob).start()

            @pl.when(s + _NBUF < total)
            def _():
                _strip_copy(s + _NBUF, b).start()
        return carry

    lax.fori_loop(0, total // _NBUF, _outer, 0)

    _out_copy(jnp.int32(total - 2), 0).wait()
    _out_copy(jnp.int32(total - 1), 1).wait()


def kernel(raw_x, adj, W1, b1, W2, b2):
    n, nfeat = raw_x.shape
    nhid = W1.shape[1]
    ncls = W2.shape[1]
    b1r = b1.reshape(1, nhid)
    b2r = b2.reshape(1, ncls)

    vmem = pl.BlockSpec(memory_space=pltpu.MemorySpace.VMEM)
    hbm = pl.BlockSpec(memory_space=pltpu.MemorySpace.HBM)

    pred = pl.pallas_call(
        _body,
        in_specs=[hbm, vmem, hbm, vmem, vmem, vmem],
        out_specs=hbm,
        out_shape=jax.ShapeDtypeStruct((n, ncls), jnp.float32),
        scratch_shapes=(
            [pltpu.VMEM((_BI, n), jnp.float32) for _ in range(_NBUF)]
            + [pltpu.VMEM((_BI, ncls), jnp.float32) for _ in range(2)]
            + [pltpu.VMEM((n, nfeat), jnp.float32),
               pltpu.VMEM((n, nhid), jnp.float32),
               pltpu.VMEM((n, ncls), jnp.float32)]
            + [pltpu.SemaphoreType.DMA for _ in range(_NBUF + 3)]
        ),
    )(raw_x, W1, adj, b1r, W2, b2r)
    return pred
